# baseline (speedup 1.0000x reference)
import functools

import jax
import jax.numpy as jnp
from jax import lax
from jax.experimental import pallas as pl
from jax.experimental.pallas import tpu as pltpu
from jax.experimental.pallas import tpu_sc as plsc

NC = 2
NS = 16
LANES = 16
IDX_CHUNK = 128
W = 48


def _build_kernel(B, L, D, S):
    NW = NC * NS
    UPW = B // NW
    PPW = UPW * L
    NSC = PPW // S
    KI = S // IDX_CHUNK
    NG = S // LANES
    assert NSC % 2 == 0 and NSC >= 4

    mesh = plsc.VectorSubcoreMesh(core_axis_name="c", subcore_axis_name="s")

    @functools.partial(
        pl.kernel,
        mesh=mesh,
        compiler_params=pltpu.CompilerParams(
            needs_layout_passes=False, use_tc_tiling_on_sc=False),
        out_type=jax.ShapeDtypeStruct((B * L,), jnp.float32),
        scratch_types=[
            pltpu.VMEM((UPW, D), jnp.float32),
            pltpu.VMEM((UPW,), jnp.float32),
            pltpu.VMEM((2, S), jnp.int32),
            pltpu.VMEM((2, S, W), jnp.float32),
            pltpu.VMEM((2, S), jnp.float32),
            pltpu.SemaphoreType.DMA,
            pltpu.SemaphoreType.DMA,
            pltpu.SemaphoreType.DMA,
            pltpu.SemaphoreType.DMA,
        ],
    )
    def mf_kernel(items_h, ifused_h, ueb_h, ubb_h, out_h,
                  ue_v, ub_v, idx_v, irows_v, out_v,
                  gsem0, gsem1, osem0, osem1):
        w = lax.axis_index("s") * NC + lax.axis_index("c")
        ubase = w * UPW
        pbase0 = w * PPW
        gsem = (gsem0, gsem1)
        osem = (osem0, osem1)

        def fire_gathers(sc, b):
            pbase = pbase0 + sc * S
            pltpu.sync_copy(items_h.at[pl.ds(pbase, S)], idx_v.at[b])
            for k in range(KI):
                sl = pl.ds(k * IDX_CHUNK, IDX_CHUNK)
                idx = idx_v.at[b, sl]
                pltpu.async_copy(ifused_h.at[idx], irows_v.at[b, sl], gsem[b])

        def drain_gathers(sc, b):
            for k in range(KI):
                sl = pl.ds(k * IDX_CHUNK, IDX_CHUNK)
                idx = idx_v.at[b, sl]
                pltpu.make_async_copy(
                    ifused_h.at[idx], irows_v.at[b, sl], gsem[b]).wait()

        pltpu.sync_copy(ueb_h.at[pl.ds(ubase, UPW)], ue_v)
        pltpu.sync_copy(ubb_h.at[pl.ds(ubase, UPW)], ub_v)
        fire_gathers(jnp.int32(0), 0)
        fire_gathers(jnp.int32(1), 1)

        lid = lax.iota(jnp.int32, 16)

        def body(sc, b):
            drain_gathers(sc, b)

            @pl.when(sc >= 2)
            def _():
                pltpu.make_async_copy(
                    out_v.at[b],
                    out_h.at[pl.ds(pbase0 + (sc - 2) * S, S)],
                    osem[b]).wait()

            def group(g, c2):
                p_local = g * LANES + lid
                p_worker = sc * S + p_local
                u_loc = lax.div(p_worker, jnp.int32(L))
                acc = plsc.load_gather(ub_v, [u_loc])
                acc = acc + plsc.load_gather(
                    irows_v.at[b], [p_local, jnp.full((16,), D, jnp.int32)])
                for d in range(D):
                    dv = jnp.full((16,), d, jnp.int32)
                    ie = plsc.load_gather(irows_v.at[b], [p_local, dv])
                    ue = plsc.load_gather(ue_v, [u_loc, dv])
                    acc = acc + ie * ue
                out_v[b, pl.ds(g * LANES, LANES)] = acc
                return c2

            lax.fori_loop(0, NG, group, 0, unroll=False)
            pltpu.async_copy(
                out_v.at[b], out_h.at[pl.ds(pbase0 + sc * S, S)], osem[b])

            @pl.when(sc + 2 < NSC)
            def _():
                fire_gathers(sc + 2, b)

        def pair_body(sc2, carry):
            body(2 * sc2, 0)
            body(2 * sc2 + 1, 1)
            return carry

        lax.fori_loop(0, NSC // 2, pair_body, 0, unroll=False)

        for b in range(2):
            pltpu.make_async_copy(
                out_v.at[b],
                out_h.at[pl.ds(pbase0 + (NSC - 2 + b) * S, S)],
                osem[b]).wait()

    return mf_kernel


def kernel(users, items, user_emb, item_emb, user_bias, item_bias, bias):
    B, L = items.shape
    N, D = item_emb.shape
    users = users.astype(jnp.int32)
    items_flat = items.astype(jnp.int32).reshape(-1)
    ifused = jnp.concatenate(
        [item_emb, item_bias[:, None],
         jnp.zeros((N, W - D - 1), jnp.float32)], axis=1)
    ue_b = jnp.take(user_emb, users, axis=0)
    ub_b = jnp.take(user_bias, users, axis=0) + bias[0]
    fn = _build_kernel(B, L, D, S=640)
    out_flat = fn(items_flat, ifused, ue_b, ub_b)
    return out_flat.reshape(B, L)


# speedup vs baseline: 1.4965x; 1.4965x over previous
import functools

import jax
import jax.numpy as jnp
from jax import lax
from jax.experimental import pallas as pl
from jax.experimental.pallas import tpu as pltpu
from jax.experimental.pallas import tpu_sc as plsc

NC = 2
NS = 16
LANES = 16
IDX_CHUNK = 128


def _build_kernel(B, L, D, NI, S):
    NW = NC * NS
    UPW = B // NW
    PPW = UPW * L
    NSC = PPW // S
    KI = S // IDX_CHUNK
    NG = S // LANES
    NIS = NI // NS
    assert NSC % 2 == 0 and NSC >= 4

    mesh = plsc.VectorSubcoreMesh(core_axis_name="c", subcore_axis_name="s")

    @functools.partial(
        pl.kernel,
        mesh=mesh,
        compiler_params=pltpu.CompilerParams(
            needs_layout_passes=False, use_tc_tiling_on_sc=False),
        out_type=jax.ShapeDtypeStruct((B * L,), jnp.float32),
        scratch_types=[
            pltpu.VMEM((UPW, D), jnp.float32),
            pltpu.VMEM((UPW,), jnp.float32),
            pltpu.VMEM((2, S), jnp.int32),
            pltpu.VMEM((2, S, D), jnp.float32),
            pltpu.VMEM((2, S), jnp.float32),
            pltpu.VMEM((2, S), jnp.float32),
            pltpu.VMEM_SHARED((NI,), jnp.float32),
            pltpu.SemaphoreType.DMA,
            pltpu.SemaphoreType.DMA,
            pltpu.SemaphoreType.DMA,
            pltpu.SemaphoreType.DMA,
        ],
    )
    def mf_kernel(items_h, iemb_h, ibias_h, ueb_h, ubb_h, out_h,
                  ue_v, ub_v, idx_v, irows_v, ibias_v, out_v, sb_v,
                  gsem0, gsem1, osem0, osem1):
        s_id = lax.axis_index("s")
        w = s_id * NC + lax.axis_index("c")
        ubase = w * UPW
        pbase0 = w * PPW
        gsem = (gsem0, gsem1)
        osem = (osem0, osem1)

        def fire_gathers(sc, b):
            pbase = pbase0 + sc * S
            pltpu.sync_copy(items_h.at[pl.ds(pbase, S)], idx_v.at[b])
            for k in range(KI):
                sl = pl.ds(k * IDX_CHUNK, IDX_CHUNK)
                idx = idx_v.at[b, sl]
                pltpu.async_copy(iemb_h.at[idx], irows_v.at[b, sl], gsem[b])

        def drain_gathers(sc, b):
            for k in range(KI):
                sl = pl.ds(k * IDX_CHUNK, IDX_CHUNK)
                idx = idx_v.at[b, sl]
                pltpu.make_async_copy(
                    iemb_h.at[idx], irows_v.at[b, sl], gsem[b]).wait()
                pltpu.sync_copy(sb_v.at[idx], ibias_v.at[b, sl])

        @pl.when(s_id == 0)
        def _():
            pltpu.sync_copy(ibias_h, sb_v)

        pltpu.sync_copy(ueb_h.at[pl.ds(ubase, UPW)], ue_v)
        pltpu.sync_copy(ubb_h.at[pl.ds(ubase, UPW)], ub_v)
        plsc.subcore_barrier()

        fire_gathers(jnp.int32(0), 0)
        fire_gathers(jnp.int32(1), 1)

        lid = lax.iota(jnp.int32, 16)

        def body(sc, b):
            drain_gathers(sc, b)

            @pl.when(sc >= 2)
            def _():
                pltpu.make_async_copy(
                    out_v.at[b],
                    out_h.at[pl.ds(pbase0 + (sc - 2) * S, S)],
                    osem[b]).wait()

            def group(g, c2):
                p_local = g * LANES + lid
                p_worker = sc * S + p_local
                u_loc = lax.div(p_worker, jnp.int32(L))
                acc = plsc.load_gather(ub_v, [u_loc])
                acc = acc + plsc.load_gather(ibias_v.at[b], [p_local])
                for d in range(D):
                    dv = jnp.full((16,), d, jnp.int32)
                    ie = plsc.load_gather(irows_v.at[b], [p_local, dv])
                    ue = plsc.load_gather(ue_v, [u_loc, dv])
                    acc = acc + ie * ue
                out_v[b, pl.ds(g * LANES, LANES)] = acc
                return c2

            lax.fori_loop(0, NG, group, 0, unroll=False)
            pltpu.async_copy(
                out_v.at[b], out_h.at[pl.ds(pbase0 + sc * S, S)], osem[b])

            @pl.when(sc + 2 < NSC)
            def _():
                fire_gathers(sc + 2, b)

        def pair_body(sc2, carry):
            body(2 * sc2, 0)
            body(2 * sc2 + 1, 1)
            return carry

        lax.fori_loop(0, NSC // 2, pair_body, 0, unroll=False)

        for b in range(2):
            pltpu.make_async_copy(
                out_v.at[b],
                out_h.at[pl.ds(pbase0 + (NSC - 2 + b) * S, S)],
                osem[b]).wait()

    return mf_kernel


def kernel(users, items, user_emb, item_emb, user_bias, item_bias, bias):
    B, L = items.shape
    NI, D = item_emb.shape
    users = users.astype(jnp.int32)
    items_flat = items.astype(jnp.int32).reshape(-1)
    ue_b = jnp.take(user_emb, users, axis=0)
    ub_b = jnp.take(user_bias, users, axis=0) + bias[0]
    fn = _build_kernel(B, L, D, NI, S=640)
    out_flat = fn(items_flat, item_emb, item_bias, ue_b, ub_b)
    return out_flat.reshape(B, L)


# speedup vs baseline: 1.5169x; 1.0137x over previous
import functools

import jax
import jax.numpy as jnp
from jax import lax
from jax.experimental import pallas as pl
from jax.experimental.pallas import tpu as pltpu
from jax.experimental.pallas import tpu_sc as plsc

NC = 2
NS = 16
LANES = 16
IDX_CHUNK = 128


def _build_kernel(B, L, D, NI, S):
    NW = NC * NS
    UPW = B // NW
    PPW = UPW * L
    NSC = PPW // S
    KI = S // IDX_CHUNK
    NG = S // LANES
    NIS = NI // NS
    assert NSC % 2 == 0 and NSC >= 4

    mesh = plsc.VectorSubcoreMesh(core_axis_name="c", subcore_axis_name="s")

    @functools.partial(
        pl.kernel,
        mesh=mesh,
        compiler_params=pltpu.CompilerParams(
            needs_layout_passes=False, use_tc_tiling_on_sc=False),
        out_type=jax.ShapeDtypeStruct((B * L,), jnp.float32),
        scratch_types=[
            pltpu.VMEM((UPW, D), jnp.float32),
            pltpu.VMEM((UPW,), jnp.float32),
            pltpu.VMEM((2, S), jnp.int32),
            pltpu.VMEM((2, S, D), jnp.float32),
            pltpu.VMEM((2, S), jnp.float32),
            pltpu.VMEM((2, S), jnp.float32),
            pltpu.VMEM_SHARED((NI,), jnp.float32),
            pltpu.SemaphoreType.DMA,
            pltpu.SemaphoreType.DMA,
            pltpu.SemaphoreType.DMA,
            pltpu.SemaphoreType.DMA,
        ],
    )
    def mf_kernel(items_h, iemb_h, ibias_h, ueb_h, ubb_h, out_h,
                  ue_v, ub_v, idx_v, irows_v, ibias_v, out_v, sb_v,
                  gsem0, gsem1, osem0, osem1):
        s_id = lax.axis_index("s")
        w = s_id * NC + lax.axis_index("c")
        ubase = w * UPW
        pbase0 = w * PPW
        gsem = (gsem0, gsem1)
        osem = (osem0, osem1)

        def fire_gathers(sc, b):
            pbase = pbase0 + sc * S
            pltpu.sync_copy(items_h.at[pl.ds(pbase, S)], idx_v.at[b])
            for k in range(KI):
                sl = pl.ds(k * IDX_CHUNK, IDX_CHUNK)
                idx = idx_v.at[b, sl]
                pltpu.async_copy(iemb_h.at[idx], irows_v.at[b, sl], gsem[b])

        def drain_gathers(sc, b):
            for k in range(KI):
                sl = pl.ds(k * IDX_CHUNK, IDX_CHUNK)
                idx = idx_v.at[b, sl]
                pltpu.make_async_copy(
                    iemb_h.at[idx], irows_v.at[b, sl], gsem[b]).wait()
            pltpu.sync_copy(sb_v.at[idx_v.at[b]], ibias_v.at[b])

        @pl.when(s_id == 0)
        def _():
            pltpu.sync_copy(ibias_h, sb_v)

        pltpu.sync_copy(ueb_h.at[pl.ds(ubase, UPW)], ue_v)
        pltpu.sync_copy(ubb_h.at[pl.ds(ubase, UPW)], ub_v)
        plsc.subcore_barrier()

        fire_gathers(jnp.int32(0), 0)
        fire_gathers(jnp.int32(1), 1)

        lid = lax.iota(jnp.int32, 16)

        def body(sc, b):
            drain_gathers(sc, b)

            @pl.when(sc >= 2)
            def _():
                pltpu.make_async_copy(
                    out_v.at[b],
                    out_h.at[pl.ds(pbase0 + (sc - 2) * S, S)],
                    osem[b]).wait()

            def group(g, c2):
                p_local = g * LANES + lid
                p_worker = sc * S + p_local
                u_loc = lax.div(p_worker, jnp.int32(L))
                acc = plsc.load_gather(ub_v, [u_loc])
                acc = acc + plsc.load_gather(ibias_v.at[b], [p_local])
                for d in range(D):
                    dv = jnp.full((16,), d, jnp.int32)
                    ie = plsc.load_gather(irows_v.at[b], [p_local, dv])
                    ue = plsc.load_gather(ue_v, [u_loc, dv])
                    acc = acc + ie * ue
                out_v[b, pl.ds(g * LANES, LANES)] = acc
                return c2

            lax.fori_loop(0, NG, group, 0, unroll=False)
            pltpu.async_copy(
                out_v.at[b], out_h.at[pl.ds(pbase0 + sc * S, S)], osem[b])

            @pl.when(sc + 2 < NSC)
            def _():
                fire_gathers(sc + 2, b)

        def pair_body(sc2, carry):
            body(2 * sc2, 0)
            body(2 * sc2 + 1, 1)
            return carry

        lax.fori_loop(0, NSC // 2, pair_body, 0, unroll=False)

        for b in range(2):
            pltpu.make_async_copy(
                out_v.at[b],
                out_h.at[pl.ds(pbase0 + (NSC - 2 + b) * S, S)],
                osem[b]).wait()

    return mf_kernel


def kernel(users, items, user_emb, item_emb, user_bias, item_bias, bias):
    B, L = items.shape
    NI, D = item_emb.shape
    users = users.astype(jnp.int32)
    items_flat = items.astype(jnp.int32).reshape(-1)
    ue_b = jnp.take(user_emb, users, axis=0)
    ub_b = jnp.take(user_bias, users, axis=0) + bias[0]
    fn = _build_kernel(B, L, D, NI, S=640)
    out_flat = fn(items_flat, item_emb, item_bias, ue_b, ub_b)
    return out_flat.reshape(B, L)
